# baseline (speedup 1.0000x reference)
import jax
import jax.numpy as jnp
from jax.experimental import pallas as pl
from jax.experimental.pallas import tpu as pltpu

_STRIDE = 8.0
_NC = 80
_ROWS = 512
_POS_STEP = _ROWS * 8


def _rot(x, s):
    return jnp.roll(x, -s, axis=1)


def _detect_body(anchors_ref, sel_ref, sel8_ref, cls_ref, bbox_ref, conf_ref,
                 pb_ref, idx_ref, confs_ref):
    t = pl.program_id(0)

    x = cls_ref[...]
    lpos = jax.lax.broadcasted_iota(jnp.int32, x.shape, 1) % _NC
    mval = x
    midx = lpos
    for s in (1, 2, 1):
        cv = _rot(mval, s)
        ci = _rot(midx, s)
        ok = (lpos < (_NC - s)) & (cv > mval)
        mval = jnp.where(ok, cv, mval)
        midx = jnp.where(ok, ci, midx)
    sel = sel_ref[...]
    mval = jnp.dot(mval, sel, preferred_element_type=jnp.float32)
    midx = jnp.dot(midx.astype(jnp.float32), sel,
                   preferred_element_type=jnp.float32)
    lp2 = jax.lax.broadcasted_iota(jnp.int32, mval.shape, 1) % 16
    for s in (1, 2, 4, 8):
        cv = _rot(mval, s)
        ci = _rot(midx, s)
        ok = (lp2 < (16 - s)) & (cv > mval)
        mval = jnp.where(ok, cv, mval)
        midx = jnp.where(ok, ci, midx)
    sel8 = sel8_ref[...]
    m8 = jnp.dot(mval, sel8, preferred_element_type=jnp.float32)
    i8 = jnp.dot(midx, sel8, preferred_element_type=jnp.float32)
    idx_ref[...] = i8.astype(jnp.int32)
    confs_ref[...] = jax.nn.sigmoid(conf_ref[...]) * jax.nn.sigmoid(m8)

    bb = bbox_ref[...]
    s4 = jax.nn.sigmoid(bb)
    rowi = jax.lax.broadcasted_iota(jnp.int32, bb.shape, 0) + t * bb.shape[0]
    lane = jax.lax.broadcasted_iota(jnp.int32, bb.shape, 1)
    flat4 = rowi * 1024 + lane
    ch = lane % 4
    pos = flat4 // 4
    w = (pos % 64).astype(jnp.float32)
    h = ((pos // 64) % 64).astype(jnp.float32)
    a = (pos // 4096) % 3
    xy = (s4 * 2.0 - 0.5 + jnp.where(ch == 0, w, h)) * _STRIDE
    aw = jnp.where(a == 0, anchors_ref[0, 0],
                   jnp.where(a == 1, anchors_ref[1, 0], anchors_ref[2, 0]))
    ah = jnp.where(a == 0, anchors_ref[0, 1],
                   jnp.where(a == 1, anchors_ref[1, 1], anchors_ref[2, 1]))
    wh = (s4 * 2.0) ** 2 * jnp.where(ch == 2, aw, ah)
    pb_ref[...] = jnp.where(ch < 2, xy, wh)


def kernel(bbox, conf, cls_logits, anchors):
    nB, nA, nH, nW, nC = cls_logits.shape
    P = nH * nW
    n = nA * P
    NPOS = nB * n
    R = NPOS * nC // 640
    RB = NPOS * 4 // 1024
    R8 = NPOS // 8

    sel = (jnp.arange(640)[:, None] == 5 * jnp.arange(128)[None, :]
           ).astype(jnp.float32)
    sel8 = (jnp.arange(128)[:, None] == 16 * jnp.arange(8)[None, :]
            ).astype(jnp.float32)

    steps = R // _ROWS
    rb = RB // steps
    pb, idx, confs = pl.pallas_call(
        _detect_body,
        grid=(steps,),
        in_specs=[
            pl.BlockSpec(memory_space=pltpu.SMEM),
            pl.BlockSpec((640, 128), lambda t: (0, 0)),
            pl.BlockSpec((128, 8), lambda t: (0, 0)),
            pl.BlockSpec((_ROWS, 640), lambda t: (t, 0)),
            pl.BlockSpec((rb, 1024), lambda t: (t, 0)),
            pl.BlockSpec((_ROWS, 8), lambda t: (t, 0)),
        ],
        out_specs=[
            pl.BlockSpec((rb, 1024), lambda t: (t, 0)),
            pl.BlockSpec((_ROWS, 8), lambda t: (t, 0)),
            pl.BlockSpec((_ROWS, 8), lambda t: (t, 0)),
        ],
        out_shape=[
            jax.ShapeDtypeStruct((RB, 1024), jnp.float32),
            jax.ShapeDtypeStruct((R8, 8), jnp.int32),
            jax.ShapeDtypeStruct((R8, 8), jnp.float32),
        ],
        compiler_params=pltpu.CompilerParams(
            dimension_semantics=("arbitrary",)),
    )(anchors, sel, sel8, cls_logits.reshape(R, 640), bbox.reshape(RB, 1024),
      conf.reshape(R8, 8))

    return (pb.reshape(nB, n, 4), idx.reshape(nB, n), confs.reshape(nB, n))
